# baseline (speedup 1.0000x reference)
import functools

import jax
import jax.numpy as jnp
from jax import lax
from jax.experimental import pallas as pl
from jax.experimental.pallas import tpu as pltpu
from jax.experimental.pallas import tpu_sc as plsc

_NUM_VOCAB = 1000000
_MAXLEN = 200
_NUM_HID = 64
_BATCH = 4096
_SEQ = 200

_NC = 2
_NS = 16
_NW = _NC * _NS
_TOTAL = _BATCH * _SEQ
_ROWS_PER_W = _TOTAL // _NW
_CHUNK = _MAXLEN
_NCHUNK = _ROWS_PER_W // _CHUNK
_LANES = 16
_SLICES = _NUM_HID // _LANES


def _body(idx2_hbm, hoff_hbm, emb2_hbm, pos_hbm, out_hbm,
          idx0, idx1, hof0, hof1, raw0, raw1, cmp0, cmp1,
          g0, g1, o0, o1, i0, i1, p0, p1):
    cid = lax.axis_index("c")
    sid = lax.axis_index("s")
    wid = sid * _NC + cid
    base = pl.multiple_of(wid * _ROWS_PER_W, _CHUNK)
    row0 = wid * _NCHUNK

    bufs = ((idx0, hof0, raw0, cmp0, g0, o0, i0, p0),
            (idx1, hof1, raw1, cmp1, g1, o1, i1, p1))

    def ifill(g, idx_v, hof_v, isem):
        off = pl.multiple_of(base + g * _CHUNK, _CHUNK)
        pltpu.async_copy(idx2_hbm.at[pl.ds(off, _CHUNK)], idx_v, isem)
        pltpu.async_copy(hoff_hbm.at[pl.ds(off, _CHUNK)],
                         hof_v.at[pl.ds(0, _CHUNK)], isem)

    def iwait(idx_v, hof_v, isem):
        pltpu.make_async_copy(idx2_hbm.at[pl.ds(base, _CHUNK)],
                              idx_v, isem).wait()
        pltpu.make_async_copy(hoff_hbm.at[pl.ds(base, _CHUNK)],
                              hof_v.at[pl.ds(0, _CHUNK)], isem).wait()

    def compact(raw_v, hof_v, cmp_v):
        @plsc.parallel_loop(0, _CHUNK, step=8, unroll=2)
        def _(t0):
            hb16 = hof_v[pl.ds(t0, _LANES)]
            for j in range(8):
                h = pl.multiple_of(hb16[j], _NUM_HID)
                t = t0 + j
                for c in range(_SLICES):
                    plsc.addupdate(
                        cmp_v.at[t, pl.ds(c * _LANES, _LANES)],
                        raw_v[t, pl.ds(h + c * _LANES, _LANES)])

    ifill(0, idx0, hof0, i0)
    ifill(1, idx1, hof1, i1)
    iwait(idx0, hof0, i0)
    pltpu.async_copy(emb2_hbm.at[idx0], raw0, g0)
    iwait(idx1, hof1, i1)
    pltpu.async_copy(emb2_hbm.at[idx1], raw1, g1)

    def step(i, carry):
        for b, (idx_v, hof_v, raw_v, cmp_v, gsem, osem, isem, psem) in (
                enumerate(bufs)):
            g = 2 * i + b

            @pl.when(g >= 2)
            def _():
                pltpu.make_async_copy(cmp_v, out_hbm.at[row0], osem).wait()

            pltpu.async_copy(pos_hbm, cmp_v, psem)

            pltpu.make_async_copy(emb2_hbm.at[idx_v], raw_v, gsem).wait()

            nxt = g + 2

            @pl.when(nxt < _NCHUNK)
            def _():
                off = pl.multiple_of(base + nxt * _CHUNK, _CHUNK)
                pltpu.async_copy(idx2_hbm.at[pl.ds(off, _CHUNK)], idx_v, isem)

            pltpu.make_async_copy(pos_hbm, cmp_v, psem).wait()
            compact(raw_v, hof_v, cmp_v)
            pltpu.async_copy(cmp_v, out_hbm.at[row0 + g], osem)

            @pl.when(nxt < _NCHUNK)
            def _():
                off = pl.multiple_of(base + nxt * _CHUNK, _CHUNK)
                pltpu.async_copy(hoff_hbm.at[pl.ds(off, _CHUNK)],
                                 hof_v.at[pl.ds(0, _CHUNK)], isem)
                iwait(idx_v, hof_v, isem)
                pltpu.async_copy(emb2_hbm.at[idx_v], raw_v, gsem)
        return carry

    lax.fori_loop(0, _NCHUNK // 2, step, 0)

    for idx_v, hof_v, raw_v, cmp_v, gsem, osem, isem, psem in bufs:
        pltpu.make_async_copy(cmp_v, out_hbm.at[row0], osem).wait()


_mesh = plsc.VectorSubcoreMesh(core_axis_name="c", subcore_axis_name="s")

_tok_kernel = functools.partial(
    pl.kernel,
    mesh=_mesh,
    compiler_params=pltpu.CompilerParams(needs_layout_passes=False,
                                         use_tc_tiling_on_sc=True),
    out_type=jax.ShapeDtypeStruct((_BATCH, _SEQ, _NUM_HID), jnp.float32),
    scratch_types=[
        pltpu.VMEM((_CHUNK,), jnp.int32),
        pltpu.VMEM((_CHUNK,), jnp.int32),
        pltpu.VMEM((_CHUNK + _LANES,), jnp.int32),
        pltpu.VMEM((_CHUNK + _LANES,), jnp.int32),
        pltpu.VMEM((_CHUNK, 2 * _NUM_HID), jnp.float32),
        pltpu.VMEM((_CHUNK, 2 * _NUM_HID), jnp.float32),
        pltpu.VMEM((_CHUNK, _NUM_HID), jnp.float32),
        pltpu.VMEM((_CHUNK, _NUM_HID), jnp.float32),
        pltpu.SemaphoreType.DMA,
        pltpu.SemaphoreType.DMA,
        pltpu.SemaphoreType.DMA,
        pltpu.SemaphoreType.DMA,
        pltpu.SemaphoreType.DMA,
        pltpu.SemaphoreType.DMA,
        pltpu.SemaphoreType.DMA,
        pltpu.SemaphoreType.DMA,
    ],
)(_body)


@jax.jit
def kernel(x, emb_table, pos_table):
    x_flat = x.reshape(-1).astype(jnp.int32)
    idx2 = x_flat >> 1
    hoff = (x_flat & 1) * _NUM_HID
    emb2 = emb_table.reshape(_NUM_VOCAB // 2, 2 * _NUM_HID)
    return _tok_kernel(idx2, hoff, emb2, pos_table)


# speedup vs baseline: 1.4613x; 1.4613x over previous
import functools

import jax
import jax.numpy as jnp
from jax import lax
from jax.experimental import pallas as pl
from jax.experimental.pallas import tpu as pltpu
from jax.experimental.pallas import tpu_sc as plsc

_NUM_VOCAB = 1000000
_MAXLEN = 200
_NUM_HID = 64
_BATCH = 4096
_SEQ = 200

_NC = 2
_NS = 16
_NW = _NC * _NS
_TOTAL = _BATCH * _SEQ
_ROWS_PER_W = _TOTAL // _NW
_CHUNK = _MAXLEN
_NCHUNK = _ROWS_PER_W // _CHUNK
_LANES = 16
_SLICES = _NUM_HID // _LANES


def _body(idx2_hbm, hoff_hbm, emb2_hbm, pos_hbm, out_hbm,
          pos_v, idx0, idx1, hof0, hof1, raw0, raw1, cmp0, cmp1,
          g0, g1, o0, o1, i0, i1):
    cid = lax.axis_index("c")
    sid = lax.axis_index("s")
    wid = sid * _NC + cid
    base = pl.multiple_of(wid * _ROWS_PER_W, _CHUNK)
    row0 = wid * _NCHUNK

    pltpu.sync_copy(pos_hbm, pos_v)

    bufs = ((idx0, hof0, raw0, cmp0, g0, o0, i0),
            (idx1, hof1, raw1, cmp1, g1, o1, i1))

    def ifill(g, idx_v, hof_v, isem):
        off = pl.multiple_of(base + g * _CHUNK, _CHUNK)
        pltpu.async_copy(idx2_hbm.at[pl.ds(off, _CHUNK)], idx_v, isem)
        pltpu.async_copy(hoff_hbm.at[pl.ds(off, _CHUNK)],
                         hof_v.at[pl.ds(0, _CHUNK)], isem)

    def iwait(idx_v, hof_v, isem):
        pltpu.make_async_copy(idx2_hbm.at[pl.ds(base, _CHUNK)],
                              idx_v, isem).wait()
        pltpu.make_async_copy(hoff_hbm.at[pl.ds(base, _CHUNK)],
                              hof_v.at[pl.ds(0, _CHUNK)], isem).wait()

    def compact(raw_v, hof_v, cmp_v):
        @plsc.parallel_loop(0, _CHUNK, step=8, unroll=2)
        def _(t0):
            hb16 = hof_v[pl.ds(t0, _LANES)]
            for j in range(8):
                h = pl.multiple_of(hb16[j], _NUM_HID)
                t = t0 + j
                for c in range(_SLICES):
                    cmp_v[t, pl.ds(c * _LANES, _LANES)] = (
                        raw_v[t, pl.ds(h + c * _LANES, _LANES)]
                        + pos_v[t, pl.ds(c * _LANES, _LANES)])

    ifill(0, idx0, hof0, i0)
    ifill(1, idx1, hof1, i1)
    iwait(idx0, hof0, i0)
    pltpu.async_copy(emb2_hbm.at[idx0], raw0, g0)
    iwait(idx1, hof1, i1)
    pltpu.async_copy(emb2_hbm.at[idx1], raw1, g1)

    def step(i, carry):
        for b, (idx_v, hof_v, raw_v, cmp_v, gsem, osem, isem) in (
                enumerate(bufs)):
            g = 2 * i + b

            @pl.when(g >= 2)
            def _():
                pltpu.make_async_copy(cmp_v, out_hbm.at[row0], osem).wait()

            pltpu.make_async_copy(emb2_hbm.at[idx_v], raw_v, gsem).wait()

            nxt = g + 2

            @pl.when(nxt < _NCHUNK)
            def _():
                off = pl.multiple_of(base + nxt * _CHUNK, _CHUNK)
                pltpu.async_copy(idx2_hbm.at[pl.ds(off, _CHUNK)], idx_v, isem)

            compact(raw_v, hof_v, cmp_v)
            pltpu.async_copy(cmp_v, out_hbm.at[row0 + g], osem)

            @pl.when(nxt < _NCHUNK)
            def _():
                off = pl.multiple_of(base + nxt * _CHUNK, _CHUNK)
                pltpu.async_copy(hoff_hbm.at[pl.ds(off, _CHUNK)],
                                 hof_v.at[pl.ds(0, _CHUNK)], isem)
                iwait(idx_v, hof_v, isem)
                pltpu.async_copy(emb2_hbm.at[idx_v], raw_v, gsem)
        return carry

    lax.fori_loop(0, _NCHUNK // 2, step, 0)

    for idx_v, hof_v, raw_v, cmp_v, gsem, osem, isem in bufs:
        pltpu.make_async_copy(cmp_v, out_hbm.at[row0], osem).wait()


_mesh = plsc.VectorSubcoreMesh(core_axis_name="c", subcore_axis_name="s")

_tok_kernel = functools.partial(
    pl.kernel,
    mesh=_mesh,
    compiler_params=pltpu.CompilerParams(needs_layout_passes=False,
                                         use_tc_tiling_on_sc=True),
    out_type=jax.ShapeDtypeStruct((_BATCH, _SEQ, _NUM_HID), jnp.float32),
    scratch_types=[
        pltpu.VMEM((_MAXLEN, _NUM_HID), jnp.float32),
        pltpu.VMEM((_CHUNK,), jnp.int32),
        pltpu.VMEM((_CHUNK,), jnp.int32),
        pltpu.VMEM((_CHUNK + _LANES,), jnp.int32),
        pltpu.VMEM((_CHUNK + _LANES,), jnp.int32),
        pltpu.VMEM((_CHUNK, 2 * _NUM_HID), jnp.float32),
        pltpu.VMEM((_CHUNK, 2 * _NUM_HID), jnp.float32),
        pltpu.VMEM((_CHUNK, _NUM_HID), jnp.float32),
        pltpu.VMEM((_CHUNK, _NUM_HID), jnp.float32),
        pltpu.SemaphoreType.DMA,
        pltpu.SemaphoreType.DMA,
        pltpu.SemaphoreType.DMA,
        pltpu.SemaphoreType.DMA,
        pltpu.SemaphoreType.DMA,
        pltpu.SemaphoreType.DMA,
    ],
)(_body)


@jax.jit
def kernel(x, emb_table, pos_table):
    x_flat = x.reshape(-1).astype(jnp.int32)
    idx2 = x_flat >> 1
    hoff = (x_flat & 1) * _NUM_HID
    emb2 = emb_table.reshape(_NUM_VOCAB // 2, 2 * _NUM_HID)
    return _tok_kernel(idx2, hoff, emb2, pos_table)


# speedup vs baseline: 1.4811x; 1.0136x over previous
import functools

import jax
import jax.numpy as jnp
from jax import lax
from jax.experimental import pallas as pl
from jax.experimental.pallas import tpu as pltpu
from jax.experimental.pallas import tpu_sc as plsc

_NUM_VOCAB = 1000000
_MAXLEN = 200
_NUM_HID = 64
_BATCH = 4096
_SEQ = 200

_NC = 2
_NS = 16
_NW = _NC * _NS
_TOTAL = _BATCH * _SEQ
_ROWS_PER_W = _TOTAL // _NW
_CHUNK = _MAXLEN
_NCHUNK = _ROWS_PER_W // _CHUNK
_LANES = 16
_SLICES = _NUM_HID // _LANES


def _body(idx2_hbm, hoff_hbm, emb2_hbm, pos_hbm, out_hbm,
          pos_v, idx0, idx1, hof0, hof1, raw0, raw1, cmp0, cmp1,
          g0, g1, o0, o1, i0, i1):
    cid = lax.axis_index("c")
    sid = lax.axis_index("s")
    wid = sid * _NC + cid
    base = pl.multiple_of(wid * _ROWS_PER_W, _CHUNK)
    row0 = wid * _NCHUNK

    pltpu.sync_copy(pos_hbm, pos_v)

    bufs = ((idx0, hof0, raw0, cmp0, g0, o0, i0),
            (idx1, hof1, raw1, cmp1, g1, o1, i1))

    def ifill(g, idx_v, hof_v, isem):
        off = pl.multiple_of(base + g * _CHUNK, _CHUNK)
        pltpu.async_copy(idx2_hbm.at[pl.ds(off, _CHUNK)], idx_v, isem)
        pltpu.async_copy(hoff_hbm.at[pl.ds(off, _CHUNK)],
                         hof_v.at[pl.ds(0, _CHUNK)], isem)

    def iwait(idx_v, hof_v, isem):
        pltpu.make_async_copy(idx2_hbm.at[pl.ds(base, _CHUNK)],
                              idx_v, isem).wait()
        pltpu.make_async_copy(hoff_hbm.at[pl.ds(base, _CHUNK)],
                              hof_v.at[pl.ds(0, _CHUNK)], isem).wait()

    def compact(raw_v, hof_v, cmp_v):
        @plsc.parallel_loop(0, _CHUNK, step=8, unroll=4)
        def _(t0):
            hb16 = hof_v[pl.ds(t0, _LANES)]
            for j in range(8):
                h = pl.multiple_of(hb16[j], _NUM_HID)
                t = t0 + j
                for c in range(_SLICES):
                    cmp_v[t, pl.ds(c * _LANES, _LANES)] = (
                        raw_v[t, pl.ds(h + c * _LANES, _LANES)]
                        + pos_v[t, pl.ds(c * _LANES, _LANES)])

    ifill(0, idx0, hof0, i0)
    ifill(1, idx1, hof1, i1)
    iwait(idx0, hof0, i0)
    pltpu.async_copy(emb2_hbm.at[idx0], raw0, g0)
    iwait(idx1, hof1, i1)
    pltpu.async_copy(emb2_hbm.at[idx1], raw1, g1)

    def step(i, carry):
        for b, (idx_v, hof_v, raw_v, cmp_v, gsem, osem, isem) in (
                enumerate(bufs)):
            g = 2 * i + b

            @pl.when(g >= 2)
            def _():
                pltpu.make_async_copy(cmp_v, out_hbm.at[row0], osem).wait()

            pltpu.make_async_copy(emb2_hbm.at[idx_v], raw_v, gsem).wait()

            nxt = g + 2

            @pl.when(nxt < _NCHUNK)
            def _():
                off = pl.multiple_of(base + nxt * _CHUNK, _CHUNK)
                pltpu.async_copy(idx2_hbm.at[pl.ds(off, _CHUNK)], idx_v, isem)

            compact(raw_v, hof_v, cmp_v)
            pltpu.async_copy(cmp_v, out_hbm.at[row0 + g], osem)

            @pl.when(nxt < _NCHUNK)
            def _():
                off = pl.multiple_of(base + nxt * _CHUNK, _CHUNK)
                pltpu.async_copy(hoff_hbm.at[pl.ds(off, _CHUNK)],
                                 hof_v.at[pl.ds(0, _CHUNK)], isem)
                iwait(idx_v, hof_v, isem)
                pltpu.async_copy(emb2_hbm.at[idx_v], raw_v, gsem)
        return carry

    lax.fori_loop(0, _NCHUNK // 2, step, 0)

    for idx_v, hof_v, raw_v, cmp_v, gsem, osem, isem in bufs:
        pltpu.make_async_copy(cmp_v, out_hbm.at[row0], osem).wait()


_mesh = plsc.VectorSubcoreMesh(core_axis_name="c", subcore_axis_name="s")

_tok_kernel = functools.partial(
    pl.kernel,
    mesh=_mesh,
    compiler_params=pltpu.CompilerParams(needs_layout_passes=False,
                                         use_tc_tiling_on_sc=True),
    out_type=jax.ShapeDtypeStruct((_BATCH, _SEQ, _NUM_HID), jnp.float32),
    scratch_types=[
        pltpu.VMEM((_MAXLEN, _NUM_HID), jnp.float32),
        pltpu.VMEM((_CHUNK,), jnp.int32),
        pltpu.VMEM((_CHUNK,), jnp.int32),
        pltpu.VMEM((_CHUNK + _LANES,), jnp.int32),
        pltpu.VMEM((_CHUNK + _LANES,), jnp.int32),
        pltpu.VMEM((_CHUNK, 2 * _NUM_HID), jnp.float32),
        pltpu.VMEM((_CHUNK, 2 * _NUM_HID), jnp.float32),
        pltpu.VMEM((_CHUNK, _NUM_HID), jnp.float32),
        pltpu.VMEM((_CHUNK, _NUM_HID), jnp.float32),
        pltpu.SemaphoreType.DMA,
        pltpu.SemaphoreType.DMA,
        pltpu.SemaphoreType.DMA,
        pltpu.SemaphoreType.DMA,
        pltpu.SemaphoreType.DMA,
        pltpu.SemaphoreType.DMA,
    ],
)(_body)


@jax.jit
def kernel(x, emb_table, pos_table):
    x_flat = x.reshape(-1).astype(jnp.int32)
    idx2 = x_flat >> 1
    hoff = (x_flat & 1) * _NUM_HID
    emb2 = emb_table.reshape(_NUM_VOCAB // 2, 2 * _NUM_HID)
    return _tok_kernel(idx2, hoff, emb2, pos_table)
